# baseline (speedup 1.0000x reference)
import functools

import jax
import jax.numpy as jnp
from jax import lax
from jax.experimental import pallas as pl
from jax.experimental.pallas import tpu as pltpu
from jax.experimental.pallas import tpu_sc as plsc

_NC, _NS = 2, 16
_NW = _NC * _NS


def _gather_rows(table, idx, chunk):
    V, D = table.shape
    B = idx.shape[0]
    assert B % (_NW * chunk) == 0
    b_per_w = B // _NW
    n_chunks = b_per_w // chunk
    mesh = plsc.VectorSubcoreMesh(core_axis_name="c", subcore_axis_name="s")

    @functools.partial(
        pl.kernel,
        mesh=mesh,
        out_type=jax.ShapeDtypeStruct((B, D), jnp.float32),
        scratch_types=[
            pltpu.VMEM((b_per_w,), jnp.int32),
            pltpu.VMEM((chunk, D), jnp.float32),
            pltpu.SemaphoreType.DMA,
        ],
    )
    def k(table_hbm, idx_hbm, out_hbm, idx_v, rows_v, sem):
        wid = lax.axis_index("s") * _NC + lax.axis_index("c")
        base = wid * b_per_w
        pltpu.sync_copy(idx_hbm.at[pl.ds(base, b_per_w)], idx_v)

        @pl.loop(0, n_chunks)
        def _(ci):
            off = ci * chunk
            pltpu.async_copy(
                table_hbm.at[idx_v.at[pl.ds(off, chunk)]], rows_v, sem
            ).wait()
            pltpu.sync_copy(rows_v, out_hbm.at[pl.ds(base + off, chunk)])

    return k(table, idx)


def kernel(input_features, project_map):
    B, C, H, W = input_features.shape
    Ho, Wo, _ = project_map.shape
    HW = H * W
    G = HW // 8

    rows = project_map[:, :, 0].astype(jnp.int32)
    cols = project_map[:, :, 1].astype(jnp.int32)
    idx = (rows * W + cols).reshape(-1)

    bas = ((idx >> 3) * 24 + (idx & 7)).reshape(G, 1, 8)
    idx_perm = (bas + (8 * jnp.arange(3, dtype=jnp.int32))[None, :, None]).reshape(-1)

    t4 = (
        input_features.reshape(3, 128, G, 8)
        .transpose(2, 0, 3, 1)
        .reshape(3 * HW, 128)
    )
    out4 = _gather_rows(t4, idx_perm, chunk=384)
    out = (
        out4.reshape(G, 3, 8, 128)
        .transpose(1, 3, 0, 2)
        .reshape(B, C, Ho, Wo)
    )
    return out


# speedup vs baseline: 3.5080x; 3.5080x over previous
import functools

import jax
import jax.numpy as jnp
from jax import lax
from jax.experimental import pallas as pl
from jax.experimental.pallas import tpu as pltpu
from jax.experimental.pallas import tpu_sc as plsc

_NC, _NS = 2, 16
_NW = _NC * _NS


def _gather_rows(table, idx, chunk):
    V, D = table.shape
    B = idx.shape[0]
    assert B % (_NW * chunk) == 0
    b_per_w = B // _NW
    n_chunks = b_per_w // chunk
    mesh = plsc.VectorSubcoreMesh(core_axis_name="c", subcore_axis_name="s")

    @functools.partial(
        pl.kernel,
        mesh=mesh,
        out_type=jax.ShapeDtypeStruct((B, D), jnp.float32),
        scratch_types=[
            pltpu.VMEM((b_per_w,), jnp.int32),
            pltpu.VMEM((chunk, D), jnp.float32),
            pltpu.SemaphoreType.DMA,
        ],
    )
    def k(table_hbm, idx_hbm, out_hbm, idx_v, rows_v, sem):
        wid = lax.axis_index("s") * _NC + lax.axis_index("c")
        base = wid * b_per_w
        pltpu.sync_copy(idx_hbm.at[pl.ds(base, b_per_w)], idx_v)

        @pl.loop(0, n_chunks)
        def _(ci):
            off = ci * chunk
            pltpu.async_copy(
                table_hbm.at[idx_v.at[pl.ds(off, chunk)]], rows_v, sem
            ).wait()
            pltpu.sync_copy(rows_v, out_hbm.at[pl.ds(base + off, chunk)])

    return k(table, idx)


def kernel(input_features, project_map):
    B, C, H, W = input_features.shape
    Ho, Wo, _ = project_map.shape
    rows = project_map[:, :, 0].astype(jnp.int32)
    cols = project_map[:, :, 1].astype(jnp.int32)
    idx = (rows * W + cols).reshape(-1)
    table = (
        input_features.reshape(B * C, H, W)
        .transpose(1, 2, 0)
        .reshape(H * W, B * C)
    )
    out_t = _gather_rows(table, idx, chunk=128)
    return out_t.reshape(Ho, Wo, B * C).transpose(2, 0, 1).reshape(B, C, Ho, Wo)


# speedup vs baseline: 3.6231x; 1.0328x over previous
import functools

import jax
import jax.numpy as jnp
from jax import lax
from jax.experimental import pallas as pl
from jax.experimental.pallas import tpu as pltpu
from jax.experimental.pallas import tpu_sc as plsc

_NC, _NS = 2, 16
_NW = _NC * _NS


def _gather_rows(table, idx, chunk):
    V, D = table.shape
    B = idx.shape[0]
    assert B % (_NW * chunk) == 0
    b_per_w = B // _NW
    n_chunks = b_per_w // chunk
    mesh = plsc.VectorSubcoreMesh(core_axis_name="c", subcore_axis_name="s")

    assert n_chunks % 2 == 0 and n_chunks >= 4

    @functools.partial(
        pl.kernel,
        mesh=mesh,
        out_type=jax.ShapeDtypeStruct((B, D), jnp.float32),
        scratch_types=[
            pltpu.VMEM((b_per_w,), jnp.int32),
            pltpu.VMEM((chunk, D), jnp.float32),
            pltpu.VMEM((chunk, D), jnp.float32),
            pltpu.SemaphoreType.DMA,
            pltpu.SemaphoreType.DMA,
            pltpu.SemaphoreType.DMA,
            pltpu.SemaphoreType.DMA,
        ],
    )
    def k(table_hbm, idx_hbm, out_hbm, idx_v, buf0, buf1, g0, g1, w0, w1):
        wid = lax.axis_index("s") * _NC + lax.axis_index("c")
        base = wid * b_per_w
        pltpu.sync_copy(idx_hbm.at[pl.ds(base, b_per_w)], idx_v)

        def start_g(ci, buf, sem):
            pltpu.async_copy(
                table_hbm.at[idx_v.at[pl.ds(ci * chunk, chunk)]], buf, sem
            )

        def wait_g(buf, sem):
            pltpu.make_async_copy(
                table_hbm.at[idx_v.at[pl.ds(0, chunk)]], buf, sem
            ).wait()

        def start_w(ci, buf, sem):
            pltpu.async_copy(buf, out_hbm.at[pl.ds(base + ci * chunk, chunk)], sem)

        def wait_w(buf, sem):
            pltpu.make_async_copy(buf, out_hbm.at[pl.ds(base, chunk)], sem).wait()

        start_g(0, buf0, g0)
        start_g(1, buf1, g1)

        @pl.loop(0, (n_chunks - 2) // 2)
        def _(k2):
            ci = 2 * k2
            wait_g(buf0, g0)
            start_w(ci, buf0, w0)
            wait_g(buf1, g1)
            start_w(ci + 1, buf1, w1)
            wait_w(buf0, w0)
            start_g(ci + 2, buf0, g0)
            wait_w(buf1, w1)
            start_g(ci + 3, buf1, g1)

        wait_g(buf0, g0)
        start_w(n_chunks - 2, buf0, w0)
        wait_g(buf1, g1)
        start_w(n_chunks - 1, buf1, w1)
        wait_w(buf0, w0)
        wait_w(buf1, w1)

    return k(table, idx)


def kernel(input_features, project_map):
    B, C, H, W = input_features.shape
    Ho, Wo, _ = project_map.shape
    rows = project_map[:, :, 0].astype(jnp.int32)
    cols = project_map[:, :, 1].astype(jnp.int32)
    idx = (rows * W + cols).reshape(-1)
    table = (
        input_features.reshape(B * C, H, W)
        .transpose(1, 2, 0)
        .reshape(H * W, B * C)
    )
    out_t = _gather_rows(table, idx, chunk=128)
    return out_t.reshape(Ho, Wo, B * C).transpose(2, 0, 1).reshape(B, C, Ho, Wo)
